# baseline (speedup 1.0000x reference)
import functools

import jax
import jax.numpy as jnp
from jax import lax
from jax.experimental import pallas as pl
from jax.experimental.pallas import tpu as pltpu
from jax.experimental.pallas import tpu_sc as plsc

EMBED = 4096
NEXP = 64
K = 8
BT = 1024

NTOK = 16384
NCHUNK = 4
CH = NTOK // NCHUNK
CB = CH // BT

NW = 32
TPW = CH // NW
LANES = 16
NG = TPW // LANES


def _gate_tc_body(x_ref, w_ref, *refs):
    logits_ref, dense_ref, logits_t_ref = refs[-3:]
    x = x_ref[...]
    w = w_ref[...]
    logits_t = jax.lax.dot_general(
        w, x, (((1,), (1,)), ((), ())), preferred_element_type=jnp.float32
    )
    logits_t_ref[...] = logits_t

    m0 = jnp.max(logits_t, axis=0, keepdims=True)
    e_t = jnp.exp(logits_t - m0)
    dense_t = e_t / jnp.sum(e_t, axis=0, keepdims=True)

    logits_ref[...] = logits_t.T
    dense_ref[...] = dense_t.T


def _router_tc_chunk(x, W, chunk, logits_io=None, dense_io=None):
    off = chunk * CB
    out_shapes = (
        jax.ShapeDtypeStruct((NTOK, NEXP), jnp.float32),
        jax.ShapeDtypeStruct((NTOK, NEXP), jnp.float32),
        jax.ShapeDtypeStruct((NEXP, CH), jnp.float32),
    )
    out_specs = (
        pl.BlockSpec((BT, NEXP), lambda i: (i + off, 0)),
        pl.BlockSpec((BT, NEXP), lambda i: (i + off, 0)),
        pl.BlockSpec((NEXP, BT), lambda i: (0, i)),
    )
    in_specs = [
        pl.BlockSpec((BT, EMBED), lambda i: (i + off, 0)),
        pl.BlockSpec((NEXP, EMBED), lambda i: (0, 0)),
    ]
    args = [x, W]
    aliases = {}
    if logits_io is not None:
        in_specs.append(pl.BlockSpec((8, NEXP), lambda i: (0, 0)))
        in_specs.append(pl.BlockSpec((8, NEXP), lambda i: (0, 0)))
        args.extend([logits_io, dense_io])
        aliases = {2: 0, 3: 1}
    return pl.pallas_call(
        _gate_tc_body,
        grid=(CB,),
        in_specs=in_specs,
        out_specs=out_specs,
        out_shape=out_shapes,
        input_output_aliases=aliases,
        compiler_params=pltpu.CompilerParams(
            dimension_semantics=("arbitrary",),
        ),
    )(*args)


def _topk_sc_body(logits_t_hbm, tw_hbm, ti_hbm, lg_v, tw_v, ti_v):
    wid = lax.axis_index("s") * 2 + lax.axis_index("c")
    base = wid * TPW
    pltpu.sync_copy(logits_t_hbm.at[:, pl.ds(base, TPW)], lg_v)

    neg_inf = jnp.full((LANES,), -jnp.inf, jnp.float32)
    zero_i = jnp.zeros((LANES,), jnp.int32)

    def group_body(g, _):
        col0 = g * LANES

        def expert_body(e, carry):
            tv = carry[:K]
            tidx = carry[K:]
            v = lg_v[e, pl.ds(col0, LANES)]
            ei = jnp.full((LANES,), e, jnp.int32)
            gt = [v > tv[j] for j in range(K)]
            new_tv = [jnp.where(gt[0], v, tv[0])]
            new_ti = [jnp.where(gt[0], ei, tidx[0])]
            for j in range(1, K):
                new_tv.append(
                    jnp.where(gt[j - 1], tv[j - 1], jnp.where(gt[j], v, tv[j]))
                )
                new_ti.append(
                    jnp.where(gt[j - 1], tidx[j - 1], jnp.where(gt[j], ei, tidx[j]))
                )
            return tuple(new_tv) + tuple(new_ti)

        init = tuple([neg_inf] * K) + tuple([zero_i] * K)
        res = lax.fori_loop(0, NEXP, expert_body, init)
        tv = res[:K]
        tidx = res[K:]

        te = [jnp.exp(t - tv[0]) for t in tv]
        tsum = functools.reduce(jnp.add, te)
        inv = 1.0 / tsum

        for j in range(K):
            tw_v[j, pl.ds(col0, LANES)] = te[j] * inv
            ti_v[j, pl.ds(col0, LANES)] = tidx[j]
        return 0

    lax.fori_loop(0, NG, group_body, 0)

    pltpu.sync_copy(tw_v, tw_hbm.at[:, pl.ds(base, TPW)])
    pltpu.sync_copy(ti_v, ti_hbm.at[:, pl.ds(base, TPW)])


_topk_sc = functools.partial(
    pl.kernel,
    mesh=plsc.VectorSubcoreMesh(core_axis_name="c", subcore_axis_name="s"),
    out_type=(
        jax.ShapeDtypeStruct((K, CH), jnp.float32),
        jax.ShapeDtypeStruct((K, CH), jnp.int32),
    ),
    scratch_types=[
        pltpu.VMEM((NEXP, TPW), jnp.float32),
        pltpu.VMEM((K, TPW), jnp.float32),
        pltpu.VMEM((K, TPW), jnp.int32),
    ],
)(_topk_sc_body)


def kernel(x, W):
    logits, dense, lt = _router_tc_chunk(x, W, 0)
    tw_parts = []
    ti_parts = []
    for c in range(1, NCHUNK):
        tw_c, ti_c = _topk_sc(lt)
        tw_parts.append(tw_c)
        ti_parts.append(ti_c)
        logits, dense, lt = _router_tc_chunk(x, W, c, logits, dense)
    tw_c, ti_c = _topk_sc(lt)
    tw_parts.append(tw_c)
    ti_parts.append(ti_c)
    tw = jnp.concatenate(tw_parts, axis=1).T
    ti = jnp.concatenate(ti_parts, axis=1).T
    return logits, dense, tw, ti


# speedup vs baseline: 1.0960x; 1.0960x over previous
import functools

import jax
import jax.numpy as jnp
from jax import lax
from jax.experimental import pallas as pl
from jax.experimental.pallas import tpu as pltpu
from jax.experimental.pallas import tpu_sc as plsc

EMBED = 4096
NEXP = 64
K = 8
BT = 1024

NTOK = 16384
NW = 32
TPW = NTOK // NW
LANES = 16
NG = TPW // LANES


def _gate_tc_body(x_ref, w_ref, logits_ref, dense_ref, logits_t_ref):
    x = x_ref[...]
    w = w_ref[...]
    logits_t = jax.lax.dot_general(
        w, x, (((1,), (1,)), ((), ())), preferred_element_type=jnp.float32
    )
    logits_t_ref[...] = logits_t

    m0 = jnp.max(logits_t, axis=0, keepdims=True)
    e_t = jnp.exp(logits_t - m0)
    dense_t = e_t / jnp.sum(e_t, axis=0, keepdims=True)

    logits_ref[...] = logits_t.T
    dense_ref[...] = dense_t.T


def _router_tc(x, W):
    n_tokens = x.shape[0]
    grid = (n_tokens // BT,)
    out_shapes = (
        jax.ShapeDtypeStruct((n_tokens, NEXP), jnp.float32),
        jax.ShapeDtypeStruct((n_tokens, NEXP), jnp.float32),
        jax.ShapeDtypeStruct((NEXP, n_tokens), jnp.float32),
    )
    out_specs = (
        pl.BlockSpec((BT, NEXP), lambda i: (i, 0)),
        pl.BlockSpec((BT, NEXP), lambda i: (i, 0)),
        pl.BlockSpec((NEXP, BT), lambda i: (0, i)),
    )
    in_specs = [
        pl.BlockSpec((BT, EMBED), lambda i: (i, 0)),
        pl.BlockSpec((NEXP, EMBED), lambda i: (0, 0)),
    ]
    return pl.pallas_call(
        _gate_tc_body,
        grid=grid,
        in_specs=in_specs,
        out_specs=out_specs,
        out_shape=out_shapes,
        compiler_params=pltpu.CompilerParams(
            dimension_semantics=("arbitrary",),
        ),
    )(x, W)


def _insert(v, ei, tv, tidx):
    gt = [v > tv[j] for j in range(K)]
    new_tv = [jnp.where(gt[0], v, tv[0])]
    new_ti = [jnp.where(gt[0], ei, tidx[0])]
    for j in range(1, K):
        new_tv.append(jnp.where(gt[j - 1], tv[j - 1], jnp.where(gt[j], v, tv[j])))
        new_ti.append(jnp.where(gt[j - 1], tidx[j - 1], jnp.where(gt[j], ei, tidx[j])))
    return new_tv, new_ti


def _topk_sc_body(logits_t_hbm, tw_hbm, ti_hbm, lg_v, tw_v, ti_v):
    wid = lax.axis_index("s") * 2 + lax.axis_index("c")
    base = wid * TPW
    pltpu.sync_copy(logits_t_hbm.at[:, pl.ds(base, TPW)], lg_v)

    neg_inf = jnp.full((LANES,), -jnp.inf, jnp.float32)
    zero_i = jnp.zeros((LANES,), jnp.int32)
    lane = lax.iota(jnp.int32, LANES)

    def pair_body(p, _):
        col0 = p * (2 * LANES)

        def expert_body(e, carry):
            tva = carry[0:K]
            tia = carry[K:2 * K]
            tvb = carry[2 * K:3 * K]
            tib = carry[3 * K:4 * K]
            ce = jnp.full((LANES,), e, jnp.int32)
            va = lg_v[e, pl.ds(col0, LANES)]
            vb = lg_v[e, pl.ds(col0 + LANES, LANES)]
            ntva, ntia = _insert(va, ce, tva, tia)
            ntvb, ntib = _insert(vb, ce, tvb, tib)
            return tuple(ntva) + tuple(ntia) + tuple(ntvb) + tuple(ntib)

        init = (tuple([neg_inf] * K) + tuple([zero_i] * K)) * 2
        res = lax.fori_loop(0, NEXP, expert_body, init)

        for half, (tv, tidx) in enumerate(
            ((res[0:K], res[K:2 * K]), (res[2 * K:3 * K], res[3 * K:4 * K]))
        ):
            te = [jnp.exp(t - tv[0]) for t in tv]
            inv = 1.0 / functools.reduce(jnp.add, te)
            c0 = col0 + half * LANES
            for j in range(K):
                tw_v[j, pl.ds(c0, LANES)] = te[j] * inv
                ti_v[j, pl.ds(c0, LANES)] = tidx[j]
        return 0

    lax.fori_loop(0, NG // 2, pair_body, 0)

    pltpu.sync_copy(tw_v, tw_hbm.at[:, pl.ds(base, TPW)])
    pltpu.sync_copy(ti_v, ti_hbm.at[:, pl.ds(base, TPW)])


_topk_sc = functools.partial(
    pl.kernel,
    mesh=plsc.VectorSubcoreMesh(core_axis_name="c", subcore_axis_name="s"),
    out_type=(
        jax.ShapeDtypeStruct((K, NTOK), jnp.float32),
        jax.ShapeDtypeStruct((K, NTOK), jnp.int32),
    ),
    scratch_types=[
        pltpu.VMEM((NEXP, TPW), jnp.float32),
        pltpu.VMEM((K, TPW), jnp.float32),
        pltpu.VMEM((K, TPW), jnp.int32),
    ],
)(_topk_sc_body)


def kernel(x, W):
    logits, dense, logits_t = _router_tc(x, W)
    tw_t, ti_t = _topk_sc(logits_t)
    return logits, dense, tw_t.T, ti_t.T


# speedup vs baseline: 1.0991x; 1.0029x over previous
import functools

import jax
import jax.numpy as jnp
from jax import lax
from jax.experimental import pallas as pl
from jax.experimental.pallas import tpu as pltpu
from jax.experimental.pallas import tpu_sc as plsc

EMBED = 4096
NEXP = 64
K = 8
BT = 1024

NTOK = 16384
NW = 32
TPW = NTOK // NW
LANES = 16
NG = TPW // LANES


def _gate_tc_body(x_ref, w_ref, logits_ref, dense_ref, logits_t_ref):
    x = x_ref[...]
    w = w_ref[...]
    logits_t = jax.lax.dot_general(
        w, x, (((1,), (1,)), ((), ())), preferred_element_type=jnp.float32
    )
    logits_t_ref[...] = logits_t

    m0 = jnp.max(logits_t, axis=0, keepdims=True)
    e_t = jnp.exp(logits_t - m0)
    dense_t = e_t / jnp.sum(e_t, axis=0, keepdims=True)

    logits_ref[...] = logits_t.T
    dense_ref[...] = dense_t.T


def _router_tc(x, W):
    n_tokens = x.shape[0]
    grid = (n_tokens // BT,)
    out_shapes = (
        jax.ShapeDtypeStruct((n_tokens, NEXP), jnp.float32),
        jax.ShapeDtypeStruct((n_tokens, NEXP), jnp.float32),
        jax.ShapeDtypeStruct((NEXP, n_tokens), jnp.float32),
    )
    out_specs = (
        pl.BlockSpec((BT, NEXP), lambda i: (i, 0)),
        pl.BlockSpec((BT, NEXP), lambda i: (i, 0)),
        pl.BlockSpec((NEXP, BT), lambda i: (0, i)),
    )
    in_specs = [
        pl.BlockSpec((BT, EMBED), lambda i: (i, 0)),
        pl.BlockSpec((NEXP, EMBED), lambda i: (0, 0)),
    ]
    return pl.pallas_call(
        _gate_tc_body,
        grid=grid,
        in_specs=in_specs,
        out_specs=out_specs,
        out_shape=out_shapes,
        compiler_params=pltpu.CompilerParams(
            dimension_semantics=("arbitrary",),
        ),
    )(x, W)


def _insert(v, ei, tv, tidx):
    gt = [v > tv[j] for j in range(K)]
    new_tv = [jnp.where(gt[0], v, tv[0])]
    new_ti = [jnp.where(gt[0], ei, tidx[0])]
    for j in range(1, K):
        new_tv.append(jnp.where(gt[j - 1], tv[j - 1], jnp.where(gt[j], v, tv[j])))
        new_ti.append(jnp.where(gt[j - 1], tidx[j - 1], jnp.where(gt[j], ei, tidx[j])))
    return new_tv, new_ti


def _topk_sc_body(logits_t_hbm, tw_hbm, ti_hbm, lg_v, tw_v, ti_v):
    wid = lax.axis_index("s") * 2 + lax.axis_index("c")
    base = wid * TPW
    pltpu.sync_copy(logits_t_hbm.at[:, pl.ds(base, TPW)], lg_v)

    neg_inf = jnp.full((LANES,), -jnp.inf, jnp.float32)
    zero_i = jnp.zeros((LANES,), jnp.int32)
    lane = lax.iota(jnp.int32, LANES)

    def pair_body(p, _):
        col0 = p * (2 * LANES)

        def expert_body(e, carry):
            tva = carry[0:K]
            tia = carry[K:2 * K]
            tvb = carry[2 * K:3 * K]
            tib = carry[3 * K:4 * K]
            ce = jnp.full((LANES,), e, jnp.int32)
            va = lg_v[e, pl.ds(col0, LANES)]
            vb = lg_v[e, pl.ds(col0 + LANES, LANES)]
            ntva, ntia = _insert(va, ce, tva, tia)
            ntvb, ntib = _insert(vb, ce, tvb, tib)
            return tuple(ntva) + tuple(ntia) + tuple(ntvb) + tuple(ntib)

        init = (tuple([neg_inf] * K) + tuple([zero_i] * K)) * 2
        res = lax.fori_loop(0, NEXP, expert_body, init, unroll=8)

        for half, (tv, tidx) in enumerate(
            ((res[0:K], res[K:2 * K]), (res[2 * K:3 * K], res[3 * K:4 * K]))
        ):
            te = [jnp.exp(t - tv[0]) for t in tv]
            inv = 1.0 / functools.reduce(jnp.add, te)
            c0 = col0 + half * LANES
            for j in range(K):
                tw_v[j, pl.ds(c0, LANES)] = te[j] * inv
                ti_v[j, pl.ds(c0, LANES)] = tidx[j]
        return 0

    lax.fori_loop(0, NG // 2, pair_body, 0)

    pltpu.sync_copy(tw_v, tw_hbm.at[:, pl.ds(base, TPW)])
    pltpu.sync_copy(ti_v, ti_hbm.at[:, pl.ds(base, TPW)])


_topk_sc = functools.partial(
    pl.kernel,
    mesh=plsc.VectorSubcoreMesh(core_axis_name="c", subcore_axis_name="s"),
    out_type=(
        jax.ShapeDtypeStruct((K, NTOK), jnp.float32),
        jax.ShapeDtypeStruct((K, NTOK), jnp.int32),
    ),
    scratch_types=[
        pltpu.VMEM((NEXP, TPW), jnp.float32),
        pltpu.VMEM((K, TPW), jnp.float32),
        pltpu.VMEM((K, TPW), jnp.int32),
    ],
)(_topk_sc_body)


def kernel(x, W):
    logits, dense, logits_t = _router_tc(x, W)
    tw_t, ti_t = _topk_sc(logits_t)
    return logits, dense, tw_t.T, ti_t.T


# speedup vs baseline: 1.0997x; 1.0005x over previous
import functools

import jax
import jax.numpy as jnp
from jax import lax
from jax.experimental import pallas as pl
from jax.experimental.pallas import tpu as pltpu
from jax.experimental.pallas import tpu_sc as plsc

EMBED = 4096
NEXP = 64
K = 8
BT = 1024

NTOK = 16384
NW = 32
TPW = NTOK // NW
LANES = 16
NG = TPW // LANES


def _gate_tc_body(x_ref, w_ref, logits_ref, dense_ref, logits_t_ref):
    x = x_ref[...]
    w = w_ref[...]
    logits_t = jax.lax.dot_general(
        w, x, (((1,), (1,)), ((), ())), preferred_element_type=jnp.float32
    )
    logits_t_ref[...] = logits_t

    m0 = jnp.max(logits_t, axis=0, keepdims=True)
    e_t = jnp.exp(logits_t - m0)
    dense_t = e_t / jnp.sum(e_t, axis=0, keepdims=True)

    logits_ref[...] = logits_t.T
    dense_ref[...] = dense_t.T


def _router_tc(x, W):
    n_tokens = x.shape[0]
    grid = (n_tokens // BT,)
    out_shapes = (
        jax.ShapeDtypeStruct((n_tokens, NEXP), jnp.float32),
        jax.ShapeDtypeStruct((n_tokens, NEXP), jnp.float32),
        jax.ShapeDtypeStruct((NEXP, n_tokens), jnp.float32),
    )
    out_specs = (
        pl.BlockSpec((BT, NEXP), lambda i: (i, 0)),
        pl.BlockSpec((BT, NEXP), lambda i: (i, 0)),
        pl.BlockSpec((NEXP, BT), lambda i: (0, i)),
    )
    in_specs = [
        pl.BlockSpec((BT, EMBED), lambda i: (i, 0)),
        pl.BlockSpec((NEXP, EMBED), lambda i: (0, 0)),
    ]
    return pl.pallas_call(
        _gate_tc_body,
        grid=grid,
        in_specs=in_specs,
        out_specs=out_specs,
        out_shape=out_shapes,
        compiler_params=pltpu.CompilerParams(
            dimension_semantics=("arbitrary",),
        ),
    )(x, W)


def _insert(v, ei, tv, tidx):
    gt = [v > tv[j] for j in range(K)]
    new_tv = [jnp.where(gt[0], v, tv[0])]
    new_ti = [jnp.where(gt[0], ei, tidx[0])]
    for j in range(1, K):
        new_tv.append(jnp.where(gt[j - 1], tv[j - 1], jnp.where(gt[j], v, tv[j])))
        new_ti.append(jnp.where(gt[j - 1], tidx[j - 1], jnp.where(gt[j], ei, tidx[j])))
    return new_tv, new_ti


def _topk_sc_body(logits_t_hbm, tw_hbm, ti_hbm, lg_v, tw_v, ti_v):
    wid = lax.axis_index("s") * 2 + lax.axis_index("c")
    base = wid * TPW
    pltpu.sync_copy(logits_t_hbm.at[:, pl.ds(base, TPW)], lg_v)

    neg_inf = jnp.full((LANES,), -jnp.inf, jnp.float32)
    zero_i = jnp.zeros((LANES,), jnp.int32)

    def pair_body(p, _):
        col0 = p * (2 * LANES)

        def expert_body(e, carry):
            tva = carry[0:K]
            tia = carry[K:2 * K]
            tvb = carry[2 * K:3 * K]
            tib = carry[3 * K:4 * K]
            ce = jnp.full((LANES,), e, jnp.int32)
            va = lg_v[e, pl.ds(col0, LANES)]
            vb = lg_v[e, pl.ds(col0 + LANES, LANES)]
            ntva, ntia = _insert(va, ce, tva, tia)
            ntvb, ntib = _insert(vb, ce, tvb, tib)
            return tuple(ntva) + tuple(ntia) + tuple(ntvb) + tuple(ntib)

        init = (tuple([neg_inf] * K) + tuple([zero_i] * K)) * 2
        res = lax.fori_loop(0, NEXP, expert_body, init, unroll=8)

        for half, (tv, tidx) in enumerate(
            ((res[0:K], res[K:2 * K]), (res[2 * K:3 * K], res[3 * K:4 * K]))
        ):
            te = [jnp.exp(t - tv[0]) for t in tv]
            inv = 1.0 / functools.reduce(jnp.add, te)
            c0 = col0 + half * LANES
            for j in range(K):
                tw_v[j, pl.ds(c0, LANES)] = te[j] * inv
                ti_v[j, pl.ds(c0, LANES)] = tidx[j]
        return 0

    lax.fori_loop(0, NG // 2, pair_body, 0)

    pltpu.sync_copy(tw_v, tw_hbm.at[:, pl.ds(base, TPW)])
    pltpu.sync_copy(ti_v, ti_hbm.at[:, pl.ds(base, TPW)])


_topk_sc = functools.partial(
    pl.kernel,
    mesh=plsc.VectorSubcoreMesh(core_axis_name="c", subcore_axis_name="s"),
    out_type=(
        jax.ShapeDtypeStruct((K, NTOK), jnp.float32),
        jax.ShapeDtypeStruct((K, NTOK), jnp.int32),
    ),
    scratch_types=[
        pltpu.VMEM((NEXP, TPW), jnp.float32),
        pltpu.VMEM((K, TPW), jnp.float32),
        pltpu.VMEM((K, TPW), jnp.int32),
    ],
)(_topk_sc_body)


def kernel(x, W):
    logits, dense, logits_t = _router_tc(x, W)
    tw_t, ti_t = _topk_sc(logits_t)
    return logits, dense, tw_t.T, ti_t.T
